# baseline (speedup 1.0000x reference)
import jax
import jax.numpy as jnp
from jax.experimental import pallas as pl
from jax.experimental.pallas import tpu as pltpu

_R = 512


def kernel(sequence, table):
    batch, hist = sequence.shape
    vocab, embed = table.shape
    n = batch * hist
    nblocks = n // _R
    idx = sequence.reshape(nblocks, 1, _R)

    def body(idx_ref, table_ref, out_ref):
        def row(r, carry):
            i = idx_ref[0, 0, r]
            out_ref[pl.ds(r, 1), :] = table_ref[pl.ds(i, 1), :]
            return carry

        jax.lax.fori_loop(0, _R, row, 0, unroll=8)

    out = pl.pallas_call(
        body,
        grid=(nblocks,),
        in_specs=[
            pl.BlockSpec((1, 1, _R), lambda i: (i, 0, 0), memory_space=pltpu.SMEM),
            pl.BlockSpec((vocab, embed), lambda i: (0, 0)),
        ],
        out_specs=pl.BlockSpec((_R, embed), lambda i: (i, 0)),
        out_shape=jax.ShapeDtypeStruct((n, embed), table.dtype),
        compiler_params=pltpu.CompilerParams(
            dimension_semantics=("parallel",),
        ),
    )(idx, table)
    return out.reshape(batch, hist, embed)


# speedup vs baseline: 1.8212x; 1.8212x over previous
import jax
import jax.numpy as jnp
from jax import lax
from jax.experimental import pallas as pl
from jax.experimental.pallas import tpu as pltpu
from jax.experimental.pallas import tpu_sc as plsc

_W = 128
_NBUF = 4
_R = 512
_SC_CHUNKS = 40


def _sc_gather(idx3, table, n_rows, embed):
    nw, nwin, _ = idx3.shape
    rows_per_worker = n_rows // nw
    nc = plsc.get_sparse_core_info().num_cores
    mesh = plsc.VectorSubcoreMesh(core_axis_name="c", subcore_axis_name="s")

    @pl.kernel(
        out_type=jax.ShapeDtypeStruct((n_rows, embed), table.dtype),
        mesh=mesh,
        scratch_types=[
            pltpu.VMEM((nwin, _W), jnp.int32),
            pltpu.VMEM((_NBUF, _W, embed), table.dtype),
        ]
        + [pltpu.SemaphoreType.DMA] * (2 * _NBUF),
    )
    def _gather_kernel(table_hbm, idx_hbm, out_hbm, idx_v, bufs, *sems):
        gsems = sems[:_NBUF]
        wsems = sems[_NBUF:]
        wid = lax.axis_index("s") * nc + lax.axis_index("c")
        base = wid * rows_per_worker

        pltpu.sync_copy(idx_hbm.at[wid], idx_v)

        for b in range(_NBUF):
            pltpu.make_async_copy(
                table_hbm.at[idx_v.at[b]], bufs.at[b], gsems[b]
            ).start()

        @pl.loop(0, nwin, step=_NBUF)
        def _(w0):
            for b in range(_NBUF):
                w = w0 + b
                pltpu.make_async_copy(
                    table_hbm.at[idx_v.at[w]], bufs.at[b], gsems[b]
                ).wait()
                dst = out_hbm.at[pl.ds(base + w * _W, _W)]
                pltpu.make_async_copy(bufs.at[b], dst, wsems[b]).start()

                nxt = w + _NBUF

                @pl.when(nxt < nwin)
                def _():
                    pltpu.make_async_copy(bufs.at[b], dst, wsems[b]).wait()
                    pltpu.make_async_copy(
                        table_hbm.at[idx_v.at[nxt]], bufs.at[b], gsems[b]
                    ).start()

        for b in range(_NBUF):
            pltpu.make_async_copy(
                bufs.at[b], out_hbm.at[pl.ds(base, _W)], wsems[b]
            ).wait()

    return _gather_kernel(table, idx3)


def _tc_gather(idx3, table, n_rows, embed):
    vocab = table.shape[0]
    nblocks = idx3.shape[0]

    def body(idx_ref, table_ref, out_ref):
        def row(r, carry):
            i = idx_ref[0, 0, r]
            out_ref[pl.ds(r, 1), :] = table_ref[pl.ds(i, 1), :]
            return carry

        jax.lax.fori_loop(0, _R, row, 0, unroll=8)

    return pl.pallas_call(
        body,
        grid=(nblocks,),
        in_specs=[
            pl.BlockSpec((1, 1, _R), lambda i: (i, 0, 0), memory_space=pltpu.SMEM),
            pl.BlockSpec((vocab, embed), lambda i: (0, 0)),
        ],
        out_specs=pl.BlockSpec((_R, embed), lambda i: (i, 0)),
        out_shape=jax.ShapeDtypeStruct((n_rows, embed), table.dtype),
        compiler_params=pltpu.CompilerParams(
            dimension_semantics=("arbitrary",),
        ),
    )(idx3, table)


def kernel(sequence, table):
    batch, hist = sequence.shape
    vocab, embed = table.shape
    n = batch * hist

    info = plsc.get_sparse_core_info()
    nw = info.num_cores * info.num_subcores

    n_sc = _SC_CHUNKS * nw * _W * _NBUF
    n_tc = n - n_sc

    flat = sequence.reshape(n)
    idx_sc = flat[:n_sc].reshape(nw, n_sc // (nw * _W), _W)
    idx_tc = flat[n_sc:].reshape(n_tc // _R, 1, _R)

    out_sc = _sc_gather(idx_sc, table, n_sc, embed)
    out_tc = _tc_gather(idx_tc, table, n_tc, embed)
    out = jnp.concatenate([out_sc, out_tc], axis=0)
    return out.reshape(batch, hist, embed)


# speedup vs baseline: 3.7685x; 2.0693x over previous
import jax
import jax.numpy as jnp
from jax import lax
from jax.experimental import pallas as pl
from jax.experimental.pallas import tpu as pltpu
from jax.experimental.pallas import tpu_sc as plsc

_W = 128
_NBUF = 5


def kernel(sequence, table):
    batch, hist = sequence.shape
    vocab, embed = table.shape
    n = batch * hist

    info = plsc.get_sparse_core_info()
    nc, ns = info.num_cores, info.num_subcores
    nw = nc * ns
    rows_per_worker = n // nw
    nwin = rows_per_worker // _W
    idx3 = sequence.reshape(nw, nwin, _W)

    mesh = plsc.VectorSubcoreMesh(core_axis_name="c", subcore_axis_name="s")

    @pl.kernel(
        out_type=jax.ShapeDtypeStruct((n, embed), table.dtype),
        mesh=mesh,
        scratch_types=[
            pltpu.VMEM((nwin, _W), jnp.int32),
            pltpu.VMEM((_NBUF, _W, embed), table.dtype),
        ]
        + [pltpu.SemaphoreType.DMA] * (2 * _NBUF),
    )
    def _gather_kernel(table_hbm, idx_hbm, out_hbm, idx_v, bufs, *sems):
        gsems = sems[:_NBUF]
        wsems = sems[_NBUF:]
        wid = lax.axis_index("s") * nc + lax.axis_index("c")
        base = wid * rows_per_worker

        pltpu.sync_copy(idx_hbm.at[wid], idx_v)

        for b in range(_NBUF):
            pltpu.make_async_copy(
                table_hbm.at[idx_v.at[b]], bufs.at[b], gsems[b]
            ).start()

        @pl.loop(0, nwin, step=_NBUF)
        def _(w0):
            for b in range(_NBUF):
                w = w0 + b
                pltpu.make_async_copy(
                    table_hbm.at[idx_v.at[w]], bufs.at[b], gsems[b]
                ).wait()
                dst = out_hbm.at[pl.ds(base + w * _W, _W)]
                pltpu.make_async_copy(bufs.at[b], dst, wsems[b]).start()

                nxt = w + _NBUF

                @pl.when(nxt < nwin)
                def _():
                    pltpu.make_async_copy(bufs.at[b], dst, wsems[b]).wait()
                    pltpu.make_async_copy(
                        table_hbm.at[idx_v.at[nxt]], bufs.at[b], gsems[b]
                    ).start()

        for b in range(_NBUF):
            pltpu.make_async_copy(
                bufs.at[b], out_hbm.at[pl.ds(base, _W)], wsems[b]
            ).wait()

    out = _gather_kernel(table, idx3)
    return out.reshape(batch, hist, embed)


# speedup vs baseline: 3.9260x; 1.0418x over previous
import jax
import jax.numpy as jnp
from jax import lax
from jax.experimental import pallas as pl
from jax.experimental.pallas import tpu as pltpu
from jax.experimental.pallas import tpu_sc as plsc

_W = 128
_NBUF = 4
_NSPM = 2


def kernel(sequence, table):
    batch, hist = sequence.shape
    vocab, embed = table.shape
    n = batch * hist

    info = plsc.get_sparse_core_info()
    nc, ns = info.num_cores, info.num_subcores
    nw = nc * ns
    rows_per_worker = n // nw
    nwin = rows_per_worker // _W
    idx3 = sequence.reshape(nw, nwin, _W)

    mesh = plsc.VectorSubcoreMesh(core_axis_name="c", subcore_axis_name="s")

    @pl.kernel(
        out_type=jax.ShapeDtypeStruct((n, embed), table.dtype),
        mesh=mesh,
        scratch_types=[
            pltpu.VMEM((nwin, _W), jnp.int32),
            pltpu.VMEM((_NBUF, _W, embed), table.dtype),
            pltpu.VMEM_SHARED((ns, _NSPM, _W, embed), table.dtype),
        ]
        + [pltpu.SemaphoreType.DMA] * (_NBUF + _NSPM),
    )
    def _gather_kernel(table_hbm, idx_hbm, out_hbm, idx_v, bufs, spm_sh, *sems):
        gsems = sems[:_NBUF]
        wsems = sems[_NBUF:]
        sid = lax.axis_index("s")
        wid = sid * nc + lax.axis_index("c")
        base = wid * rows_per_worker
        spm = spm_sh.at[sid]

        pltpu.sync_copy(idx_hbm.at[wid], idx_v)

        for b in range(_NBUF):
            pltpu.make_async_copy(
                table_hbm.at[idx_v.at[b]], bufs.at[b], gsems[b]
            ).start()

        @pl.loop(0, nwin, step=_NBUF)
        def _(w0):
            for b in range(_NBUF):
                w = w0 + b
                s = b % _NSPM
                pltpu.make_async_copy(
                    table_hbm.at[idx_v.at[w]], bufs.at[b], gsems[b]
                ).wait()

                nxt = w + _NBUF
                dst = out_hbm.at[pl.ds(base + w * _W, _W)]

                @pl.when(w >= _NSPM)
                def _():
                    pltpu.make_async_copy(spm.at[s], dst, wsems[s]).wait()

                pltpu.sync_copy(bufs.at[b], spm.at[s])
                pltpu.make_async_copy(spm.at[s], dst, wsems[s]).start()

                @pl.when(nxt < nwin)
                def _():
                    pltpu.make_async_copy(
                        table_hbm.at[idx_v.at[nxt]], bufs.at[b], gsems[b]
                    ).start()

        for s in range(_NSPM):
            pltpu.make_async_copy(
                spm.at[s], out_hbm.at[pl.ds(base, _W)], wsems[s]
            ).wait()

    out = _gather_kernel(table, idx3)
    return out.reshape(batch, hist, embed)
